# Initial kernel scaffold; baseline (speedup 1.0000x reference)
import functools

import jax
import jax.numpy as jnp
from jax import lax
from jax.experimental import pallas as pl
from jax.experimental.pallas import tpu as pltpu
from jax.experimental.pallas import tpu_sc as plsc

N = 10000
E = 320000
D_IN = 128
HID = 16
D_OUT = 128

NP = 10240
ROWS_PER_TILE = NP // 16
NW = 32
CHUNK = 128
CHUNKS_PER_W = 82
EPW = CHUNK * CHUNKS_PER_W
EP = NW * EPW

_mesh = plsc.VectorSubcoreMesh(core_axis_name="c", subcore_axis_name="s")


def _zero_fill(ref, rows):
    z = jnp.zeros((16,), jnp.float32)

    def body(i, _):
        ref[i] = z
        return 0

    lax.fori_loop(0, rows, body, 0)


@functools.partial(
    pl.kernel,
    mesh=_mesh,
    out_type=jax.ShapeDtypeStruct((2, NP, 16), jnp.float32),
    scratch_types=[
        pltpu.VMEM((CHUNKS_PER_W, CHUNK), jnp.int32),
        pltpu.VMEM((CHUNK, 16), jnp.float32),
        pltpu.VMEM((ROWS_PER_TILE, 16), jnp.float32),
        pltpu.VMEM_SHARED((NP, 16), jnp.float32),
    ],
)
def _sc_count(dst_hbm, out_hbm, dst_v, ones_v, zslab_v, acc):
    c = lax.axis_index("c")
    s = lax.axis_index("s")
    wid = s * 2 + c

    pltpu.sync_copy(dst_hbm.at[wid], dst_v)

    one = jnp.full((16,), 1.0, jnp.float32)

    def fill_ones(i, _):
        ones_v[i] = one
        return 0

    lax.fori_loop(0, CHUNK, fill_ones, 0)

    _zero_fill(zslab_v, ROWS_PER_TILE)
    pltpu.sync_copy(zslab_v, acc.at[pl.ds(s * ROWS_PER_TILE, ROWS_PER_TILE)])
    plsc.subcore_barrier()

    def body(j, _):
        pltpu.sync_copy(ones_v, acc.at[dst_v.at[j]], add=True)
        return 0

    lax.fori_loop(0, CHUNKS_PER_W, body, 0)

    plsc.subcore_barrier()
    sl = pl.ds(s * ROWS_PER_TILE, ROWS_PER_TILE)
    pltpu.sync_copy(acc.at[sl], out_hbm.at[c, sl])


@functools.partial(
    pl.kernel,
    mesh=_mesh,
    out_type=jax.ShapeDtypeStruct((2, NP, 16), jnp.float32),
    scratch_types=[
        pltpu.VMEM((CHUNKS_PER_W, CHUNK), jnp.int32),
        pltpu.VMEM((CHUNKS_PER_W, CHUNK), jnp.int32),
        pltpu.VMEM((CHUNK, 16), jnp.float32),
        pltpu.VMEM((CHUNK, 16), jnp.float32),
        pltpu.VMEM((ROWS_PER_TILE, 16), jnp.float32),
        pltpu.VMEM_SHARED((NP, 16), jnp.float32),
        pltpu.SemaphoreType.DMA,
        pltpu.SemaphoreType.DMA,
    ],
)
def _sc_prop(g_hbm, src_hbm, dst_hbm, out_hbm,
             src_v, dst_v, row0, row1, zslab_v, acc, sem0, sem1):
    c = lax.axis_index("c")
    s = lax.axis_index("s")
    wid = s * 2 + c

    pltpu.sync_copy(src_hbm.at[wid], src_v)
    pltpu.sync_copy(dst_hbm.at[wid], dst_v)
    _zero_fill(zslab_v, ROWS_PER_TILE)
    pltpu.sync_copy(zslab_v, acc.at[pl.ds(s * ROWS_PER_TILE, ROWS_PER_TILE)])
    plsc.subcore_barrier()

    bufs = (row0, row1)
    sems = (sem0, sem1)

    pltpu.async_copy(g_hbm.at[src_v.at[0]], row0, sem0)
    pltpu.async_copy(g_hbm.at[src_v.at[1]], row1, sem1)

    def outer(t, _):
        j = t * 2
        for b in range(2):
            jj = j + b
            pltpu.make_async_copy(g_hbm.at[src_v.at[jj]], bufs[b], sems[b]).wait()

            @pl.when(jj + 2 < CHUNKS_PER_W)
            def _start():
                pltpu.async_copy(g_hbm.at[src_v.at[jj + 2]], bufs[b], sems[b])

            pltpu.sync_copy(bufs[b], acc.at[dst_v.at[jj]], add=True)
        return 0

    lax.fori_loop(0, CHUNKS_PER_W // 2, outer, 0)

    plsc.subcore_barrier()
    sl = pl.ds(s * ROWS_PER_TILE, ROWS_PER_TILE)
    pltpu.sync_copy(acc.at[sl], out_hbm.at[c, sl])


def _tc_mm1(x_ref, w_ref, o_ref):
    o_ref[:] = jnp.dot(x_ref[:], w_ref[:], preferred_element_type=jnp.float32)


def _tc_scale1(h1_ref, cnt_ref, og_ref, od_ref):
    deg = cnt_ref[0, :, 0:1] + cnt_ref[1, :, 0:1]
    dinv = lax.rsqrt(deg)
    od_ref[:] = jnp.broadcast_to(dinv, (NP, 16))
    og_ref[:] = h1_ref[:] * dinv


def _tc_scale2(sp_ref, dinv_ref, b1_ref, o_ref):
    s = sp_ref[0] + sp_ref[1]
    h = jnp.maximum(dinv_ref[:] * s + b1_ref[:], 0.0)
    o_ref[:] = dinv_ref[:] * h


def _tc_mm2(sp_ref, dinv_ref, w_ref, b_ref, o_ref):
    a = dinv_ref[:] * (sp_ref[0] + sp_ref[1])
    o_ref[:] = jnp.dot(a, w_ref[:], preferred_element_type=jnp.float32) + b_ref[:]


def kernel(x, edge_index, W1, b1, W2, b2):
    f32 = jnp.float32

    loop = jnp.arange(N, dtype=jnp.int32)
    pad = jnp.full((EP - E - N,), N, dtype=jnp.int32)
    src_ext = jnp.concatenate([edge_index[0], loop, pad]).reshape(NW, CHUNKS_PER_W, CHUNK)
    dst_ext = jnp.concatenate([edge_index[1], loop, pad]).reshape(NW, CHUNKS_PER_W, CHUNK)
    x_pad = jnp.zeros((NP, D_IN), f32).at[:N].set(x)

    counts = _sc_count(dst_ext)
    h1 = pl.pallas_call(
        _tc_mm1,
        out_shape=jax.ShapeDtypeStruct((NP, HID), f32),
    )(x_pad, W1)

    g1, dinv16 = pl.pallas_call(
        _tc_scale1,
        out_shape=(
            jax.ShapeDtypeStruct((NP, HID), f32),
            jax.ShapeDtypeStruct((NP, HID), f32),
        ),
    )(h1, counts)

    s1 = _sc_prop(g1, src_ext, dst_ext)

    g2 = pl.pallas_call(
        _tc_scale2,
        out_shape=jax.ShapeDtypeStruct((NP, HID), f32),
    )(s1, dinv16, b1.reshape(1, HID))

    s2 = _sc_prop(g2, src_ext, dst_ext)

    out = pl.pallas_call(
        _tc_mm2,
        out_shape=jax.ShapeDtypeStruct((NP, D_OUT), f32),
    )(s2, dinv16, W2, b2.reshape(1, D_OUT))

    return out[:N]


# speedup vs baseline: 41.0005x; 41.0005x over previous
import functools

import jax
import jax.numpy as jnp
from jax import lax
from jax.experimental import pallas as pl
from jax.experimental.pallas import tpu as pltpu
from jax.experimental.pallas import tpu_sc as plsc

N = 10000
E = 320000
D_IN = 128
HID = 16
D_OUT = 128

NP = 10240
ROWS_PER_TILE = NP // 16
NW = 32
CHUNK = 128
CHUNKS_PER_W = 82
EPW = CHUNK * CHUNKS_PER_W
EP = NW * EPW

_mesh = plsc.VectorSubcoreMesh(core_axis_name="c", subcore_axis_name="s")


def _zero_fill(ref, rows):
    z = jnp.zeros((16,), jnp.float32)

    def body(i, _):
        ref[i] = z
        return 0

    lax.fori_loop(0, rows, body, 0)


@functools.partial(
    pl.kernel,
    mesh=_mesh,
    compiler_params=pltpu.CompilerParams(use_tc_tiling_on_sc=False),
    out_type=jax.ShapeDtypeStruct((2, NP, 16), jnp.float32),
    scratch_types=[
        pltpu.VMEM((CHUNKS_PER_W, CHUNK), jnp.int32),
        pltpu.VMEM((CHUNK, 16), jnp.float32),
        pltpu.VMEM((ROWS_PER_TILE, 16), jnp.float32),
        pltpu.VMEM_SHARED((NP, 16), jnp.float32),
    ],
)
def _sc_count(dst_hbm, out_hbm, dst_v, ones_v, zslab_v, acc):
    c = lax.axis_index("c")
    s = lax.axis_index("s")
    wid = s * 2 + c

    pltpu.sync_copy(dst_hbm.at[wid], dst_v)

    one = jnp.full((16,), 1.0, jnp.float32)

    def fill_ones(i, _):
        ones_v[i] = one
        return 0

    lax.fori_loop(0, CHUNK, fill_ones, 0)

    _zero_fill(zslab_v, ROWS_PER_TILE)
    pltpu.sync_copy(zslab_v, acc.at[pl.ds(s * ROWS_PER_TILE, ROWS_PER_TILE)])
    plsc.subcore_barrier()

    def body(j, _):
        pltpu.sync_copy(ones_v, acc.at[dst_v.at[j]], add=True)
        return 0

    lax.fori_loop(0, CHUNKS_PER_W, body, 0)

    plsc.subcore_barrier()
    sl = pl.ds(s * ROWS_PER_TILE, ROWS_PER_TILE)
    pltpu.sync_copy(acc.at[sl], out_hbm.at[c, sl])


@functools.partial(
    pl.kernel,
    mesh=_mesh,
    compiler_params=pltpu.CompilerParams(use_tc_tiling_on_sc=False),
    out_type=jax.ShapeDtypeStruct((2, NP, 16), jnp.float32),
    scratch_types=[
        pltpu.VMEM((CHUNKS_PER_W, CHUNK), jnp.int32),
        pltpu.VMEM((CHUNKS_PER_W, CHUNK), jnp.int32),
        pltpu.VMEM((CHUNK, 16), jnp.float32),
        pltpu.VMEM((CHUNK, 16), jnp.float32),
        pltpu.VMEM((ROWS_PER_TILE, 16), jnp.float32),
        pltpu.VMEM_SHARED((NP, 16), jnp.float32),
        pltpu.SemaphoreType.DMA,
        pltpu.SemaphoreType.DMA,
    ],
)
def _sc_prop(g_hbm, src_hbm, dst_hbm, out_hbm,
             src_v, dst_v, row0, row1, zslab_v, acc, sem0, sem1):
    c = lax.axis_index("c")
    s = lax.axis_index("s")
    wid = s * 2 + c

    pltpu.sync_copy(src_hbm.at[wid], src_v)
    pltpu.sync_copy(dst_hbm.at[wid], dst_v)
    _zero_fill(zslab_v, ROWS_PER_TILE)
    pltpu.sync_copy(zslab_v, acc.at[pl.ds(s * ROWS_PER_TILE, ROWS_PER_TILE)])
    plsc.subcore_barrier()

    bufs = (row0, row1)
    sems = (sem0, sem1)

    pltpu.async_copy(g_hbm.at[src_v.at[0]], row0, sem0)
    pltpu.async_copy(g_hbm.at[src_v.at[1]], row1, sem1)

    def outer(t, _):
        j = t * 2
        for b in range(2):
            jj = j + b
            pltpu.make_async_copy(g_hbm.at[src_v.at[jj]], bufs[b], sems[b]).wait()

            @pl.when(jj + 2 < CHUNKS_PER_W)
            def _start():
                pltpu.async_copy(g_hbm.at[src_v.at[jj + 2]], bufs[b], sems[b])

            pltpu.sync_copy(bufs[b], acc.at[dst_v.at[jj]], add=True)
        return 0

    lax.fori_loop(0, CHUNKS_PER_W // 2, outer, 0)

    plsc.subcore_barrier()
    sl = pl.ds(s * ROWS_PER_TILE, ROWS_PER_TILE)
    pltpu.sync_copy(acc.at[sl], out_hbm.at[c, sl])


def _tc_mm1(x_ref, w_ref, o_ref):
    o_ref[:] = jnp.dot(x_ref[:], w_ref[:], preferred_element_type=jnp.float32)


def _tc_scale1(h1_ref, cnt_ref, og_ref, od_ref):
    deg = cnt_ref[0, :, 0:1] + cnt_ref[1, :, 0:1]
    dinv = lax.rsqrt(deg)
    od_ref[:] = jnp.broadcast_to(dinv, (NP, 16))
    og_ref[:] = h1_ref[:] * dinv


def _tc_scale2(sp_ref, dinv_ref, b1_ref, o_ref):
    s = sp_ref[0] + sp_ref[1]
    h = jnp.maximum(dinv_ref[:] * s + b1_ref[:], 0.0)
    o_ref[:] = dinv_ref[:] * h


def _tc_mm2(sp_ref, dinv_ref, w_ref, b_ref, o_ref):
    a = dinv_ref[:] * (sp_ref[0] + sp_ref[1])
    o_ref[:] = jnp.dot(a, w_ref[:], preferred_element_type=jnp.float32) + b_ref[:]


def kernel(x, edge_index, W1, b1, W2, b2):
    f32 = jnp.float32

    loop = jnp.arange(N, dtype=jnp.int32)
    pad = jnp.full((EP - E - N,), N, dtype=jnp.int32)
    src_ext = jnp.concatenate([edge_index[0], loop, pad]).reshape(NW, CHUNKS_PER_W, CHUNK)
    dst_ext = jnp.concatenate([edge_index[1], loop, pad]).reshape(NW, CHUNKS_PER_W, CHUNK)
    x_pad = jnp.zeros((NP, D_IN), f32).at[:N].set(x)

    counts = _sc_count(dst_ext)
    h1 = pl.pallas_call(
        _tc_mm1,
        out_shape=jax.ShapeDtypeStruct((NP, HID), f32),
    )(x_pad, W1)

    g1, dinv16 = pl.pallas_call(
        _tc_scale1,
        out_shape=(
            jax.ShapeDtypeStruct((NP, HID), f32),
            jax.ShapeDtypeStruct((NP, HID), f32),
        ),
    )(h1, counts)

    s1 = _sc_prop(g1, src_ext, dst_ext)

    g2 = pl.pallas_call(
        _tc_scale2,
        out_shape=jax.ShapeDtypeStruct((NP, HID), f32),
    )(s1, dinv16, b1.reshape(1, HID))

    s2 = _sc_prop(g2, src_ext, dst_ext)

    out = pl.pallas_call(
        _tc_mm2,
        out_shape=jax.ShapeDtypeStruct((NP, D_OUT), f32),
    )(s2, dinv16, W2, b2.reshape(1, D_OUT))

    return out[:N]


# speedup vs baseline: 54.6371x; 1.3326x over previous
import functools

import jax
import jax.numpy as jnp
from jax import lax
from jax.experimental import pallas as pl
from jax.experimental.pallas import tpu as pltpu
from jax.experimental.pallas import tpu_sc as plsc

N = 10000
E = 320000
D_IN = 128
HID = 16
D_OUT = 128

NP = 10240
ROWS_PER_TILE = NP // 16
NW = 32
CHUNK = 128
CHUNKS_PER_W = 82
EPW = CHUNK * CHUNKS_PER_W
EP = NW * EPW

_mesh = plsc.VectorSubcoreMesh(core_axis_name="c", subcore_axis_name="s")


def _zero_fill(ref, rows):
    z = jnp.zeros((16,), jnp.float32)

    def body(i, _):
        ref[i] = z
        return 0

    lax.fori_loop(0, rows, body, 0)


@functools.partial(
    pl.kernel,
    mesh=_mesh,
    compiler_params=pltpu.CompilerParams(use_tc_tiling_on_sc=False),
    out_type=jax.ShapeDtypeStruct((2, NP, 16), jnp.float32),
    scratch_types=[
        pltpu.VMEM((CHUNKS_PER_W, CHUNK), jnp.int32),
        pltpu.VMEM((CHUNK, 16), jnp.float32),
        pltpu.VMEM((ROWS_PER_TILE, 16), jnp.float32),
        pltpu.VMEM_SHARED((NP, 16), jnp.float32),
    ],
)
def _sc_count(dst_hbm, out_hbm, dst_v, ones_v, zslab_v, acc):
    c = lax.axis_index("c")
    s = lax.axis_index("s")
    wid = s * 2 + c

    pltpu.sync_copy(dst_hbm.at[wid], dst_v)

    one = jnp.full((16,), 1.0, jnp.float32)

    def fill_ones(i, _):
        ones_v[i] = one
        return 0

    lax.fori_loop(0, CHUNK, fill_ones, 0)

    _zero_fill(zslab_v, ROWS_PER_TILE)
    pltpu.sync_copy(zslab_v, acc.at[pl.ds(s * ROWS_PER_TILE, ROWS_PER_TILE)])
    plsc.subcore_barrier()

    def body(j, _):
        pltpu.sync_copy(ones_v, acc.at[dst_v.at[j]], add=True)
        return 0

    lax.fori_loop(0, CHUNKS_PER_W, body, 0)

    plsc.subcore_barrier()
    sl = pl.ds(s * ROWS_PER_TILE, ROWS_PER_TILE)
    pltpu.sync_copy(acc.at[sl], out_hbm.at[c, sl])


@functools.partial(
    pl.kernel,
    mesh=_mesh,
    compiler_params=pltpu.CompilerParams(use_tc_tiling_on_sc=False),
    out_type=jax.ShapeDtypeStruct((2, NP, 16), jnp.float32),
    scratch_types=[
        pltpu.VMEM((CHUNKS_PER_W, CHUNK), jnp.int32),
        pltpu.VMEM((CHUNKS_PER_W, CHUNK), jnp.int32),
        pltpu.VMEM((CHUNK, 16), jnp.float32),
        pltpu.VMEM((CHUNK, 16), jnp.float32),
        pltpu.VMEM((ROWS_PER_TILE, 16), jnp.float32),
        pltpu.VMEM_SHARED((NP, 16), jnp.float32),
        pltpu.VMEM_SHARED((NP, 16), jnp.float32),
        pltpu.SemaphoreType.DMA,
        pltpu.SemaphoreType.DMA,
    ],
)
def _sc_prop(g_hbm, src_hbm, dst_hbm, out_hbm,
             src_v, dst_v, row0, row1, zslab_v, acc, gtab, sem0, sem1):
    c = lax.axis_index("c")
    s = lax.axis_index("s")
    wid = s * 2 + c

    pltpu.sync_copy(src_hbm.at[wid], src_v)
    pltpu.sync_copy(dst_hbm.at[wid], dst_v)
    _zero_fill(zslab_v, ROWS_PER_TILE)
    stage = pl.ds(s * ROWS_PER_TILE, ROWS_PER_TILE)
    pltpu.sync_copy(zslab_v, acc.at[stage])
    pltpu.sync_copy(g_hbm.at[stage], gtab.at[stage])
    plsc.subcore_barrier()

    bufs = (row0, row1)
    sems = (sem0, sem1)

    pltpu.async_copy(gtab.at[src_v.at[0]], row0, sem0)
    pltpu.async_copy(gtab.at[src_v.at[1]], row1, sem1)

    def outer(t, _):
        j = t * 2
        for b in range(2):
            jj = j + b
            pltpu.make_async_copy(gtab.at[src_v.at[jj]], bufs[b], sems[b]).wait()

            @pl.when(jj + 2 < CHUNKS_PER_W)
            def _start():
                pltpu.async_copy(gtab.at[src_v.at[jj + 2]], bufs[b], sems[b])

            pltpu.sync_copy(bufs[b], acc.at[dst_v.at[jj]], add=True)
        return 0

    lax.fori_loop(0, CHUNKS_PER_W // 2, outer, 0)

    plsc.subcore_barrier()
    sl = pl.ds(s * ROWS_PER_TILE, ROWS_PER_TILE)
    pltpu.sync_copy(acc.at[sl], out_hbm.at[c, sl])


def _tc_mm1(x_ref, w_ref, o_ref):
    o_ref[:] = jnp.dot(x_ref[:], w_ref[:], preferred_element_type=jnp.float32)


def _tc_scale1(h1_ref, cnt_ref, og_ref, od_ref):
    deg = cnt_ref[0, :, 0:1] + cnt_ref[1, :, 0:1]
    dinv = lax.rsqrt(deg)
    od_ref[:] = jnp.broadcast_to(dinv, (NP, 16))
    og_ref[:] = h1_ref[:] * dinv


def _tc_scale2(sp_ref, dinv_ref, b1_ref, o_ref):
    s = sp_ref[0] + sp_ref[1]
    h = jnp.maximum(dinv_ref[:] * s + b1_ref[:], 0.0)
    o_ref[:] = dinv_ref[:] * h


def _tc_mm2(sp_ref, dinv_ref, w_ref, b_ref, o_ref):
    a = dinv_ref[:] * (sp_ref[0] + sp_ref[1])
    o_ref[:] = jnp.dot(a, w_ref[:], preferred_element_type=jnp.float32) + b_ref[:]


def kernel(x, edge_index, W1, b1, W2, b2):
    f32 = jnp.float32

    loop = jnp.arange(N, dtype=jnp.int32)
    pad = jnp.full((EP - E - N,), N, dtype=jnp.int32)
    src_ext = jnp.concatenate([edge_index[0], loop, pad]).reshape(NW, CHUNKS_PER_W, CHUNK)
    dst_ext = jnp.concatenate([edge_index[1], loop, pad]).reshape(NW, CHUNKS_PER_W, CHUNK)
    x_pad = jnp.zeros((NP, D_IN), f32).at[:N].set(x)

    counts = _sc_count(dst_ext)
    h1 = pl.pallas_call(
        _tc_mm1,
        out_shape=jax.ShapeDtypeStruct((NP, HID), f32),
    )(x_pad, W1)

    g1, dinv16 = pl.pallas_call(
        _tc_scale1,
        out_shape=(
            jax.ShapeDtypeStruct((NP, HID), f32),
            jax.ShapeDtypeStruct((NP, HID), f32),
        ),
    )(h1, counts)

    s1 = _sc_prop(g1, src_ext, dst_ext)

    g2 = pl.pallas_call(
        _tc_scale2,
        out_shape=jax.ShapeDtypeStruct((NP, HID), f32),
    )(s1, dinv16, b1.reshape(1, HID))

    s2 = _sc_prop(g2, src_ext, dst_ext)

    out = pl.pallas_call(
        _tc_mm2,
        out_shape=jax.ShapeDtypeStruct((NP, D_OUT), f32),
    )(s2, dinv16, W2, b2.reshape(1, D_OUT))

    return out[:N]
